# baseline (speedup 1.0000x reference)
import functools

import jax
import jax.numpy as jnp
from jax import lax
from jax.experimental import pallas as pl
from jax.experimental.pallas import tpu as pltpu
from jax.experimental.pallas import tpu_sc as plsc

B = 4096
S = 200
D = 64
L = 16
CHUNK = 40
CPR = S // CHUNK
NW = 32
BPW = B // NW
TASKS = BPW * CPR
DV = D // L


def _pool_body(tok1, tok2, tok3, emb1, emb2, emb3, out, idx_v, rows0, rows1, stage, sem):
    cid = lax.axis_index("c")
    sid = lax.axis_index("s")
    wid = sid * 2 + cid

    def zbody(i, carry):
        z = jnp.zeros((L,), jnp.float32)
        for j in range(3 * D // L):
            stage[i, pl.ds(L * j, L)] = z
        return carry

    lax.fori_loop(0, BPW, zbody, 0)

    for t, (tok, emb) in enumerate(((tok1, emb1), (tok2, emb2), (tok3, emb3))):
        pltpu.sync_copy(tok.at[pl.ds(wid * TASKS, TASKS)], idx_v)

        def fire(k, rbuf, emb=emb):
            pltpu.make_async_copy(emb.at[idx_v.at[k]], rbuf, sem).start()

        def drain(k, rbuf, emb=emb):
            pltpu.make_async_copy(emb.at[idx_v.at[k]], rbuf, sem).wait()

        def accum(k, rbuf, t=t):
            acc = [jnp.zeros((L,), jnp.float32) for _ in range(2 * DV)]
            for s in range(CHUNK):
                bank = (s % 2) * DV
                for j in range(DV):
                    acc[bank + j] = acc[bank + j] + rbuf[s, pl.ds(L * j, L)]
            b_loc = k // CPR
            for j in range(DV):
                plsc.addupdate(
                    stage.at[b_loc, pl.ds(t * D + L * j, L)], acc[j] + acc[DV + j]
                )

        fire(0, rows0)

        def lbody(kk, carry):
            k0 = 2 * kk
            fire(k0 + 1, rows1)
            drain(k0, rows0)
            accum(k0, rows0)

            @pl.when(kk < TASKS // 2 - 1)
            def _():
                fire(k0 + 2, rows0)

            drain(k0 + 1, rows1)
            accum(k0 + 1, rows1)
            return carry

        lax.fori_loop(0, TASKS // 2, lbody, 0)

    pltpu.sync_copy(stage, out.at[pl.ds(wid * BPW, BPW)])


_pool = functools.partial(
    pl.kernel,
    out_type=jax.ShapeDtypeStruct((B, 3 * D), jnp.float32),
    mesh=plsc.VectorSubcoreMesh(core_axis_name="c", subcore_axis_name="s"),
    scratch_types=[
        pltpu.VMEM((TASKS, CHUNK), jnp.int32),
        pltpu.VMEM((CHUNK, 2 * D), jnp.float32),
        pltpu.VMEM((CHUNK, 2 * D), jnp.float32),
        pltpu.VMEM((BPW, 3 * D), jnp.float32),
        pltpu.SemaphoreType.DMA,
    ],
    )(_pool_body)


def _mlp_body(x_ref, w1_ref, b1_ref, w2_ref, b2_ref, o_ref):
    x = x_ref[...]
    h = lax.dot_general(
        x, w1_ref[...], (((1,), (0,)), ((), ())),
        preferred_element_type=jnp.float32, precision=lax.Precision.HIGHEST,
    )
    h = jnp.maximum(h * (1.0 / S) + b1_ref[...], 0.0)
    o = lax.dot_general(
        h, w2_ref[...], (((1,), (0,)), ((), ())),
        preferred_element_type=jnp.float32, precision=lax.Precision.HIGHEST,
    )
    o_ref[...] = o + b2_ref[...]


def _mlp(pooled, W1, b1, W2, b2):
    return pl.pallas_call(
        _mlp_body,
        out_shape=jax.ShapeDtypeStruct((B, W2.shape[1]), jnp.float32),
    )(pooled, W1, b1.reshape(1, -1), W2, b2.reshape(1, -1))


def kernel(tokens_1gram, tokens_2gram, tokens_3gram, emb1, emb2, emb3, W1, b1, W2, b2):
    t1 = (tokens_1gram >> 1).reshape(-1, CHUNK)
    t2 = (tokens_2gram >> 1).reshape(-1, CHUNK)
    t3 = (tokens_3gram >> 1).reshape(-1, CHUNK)
    pooled = _pool(t1, t2, t3, emb1.reshape(-1, 128), emb2.reshape(-1, 128), emb3.reshape(-1, 128))
    return _mlp(pooled, W1, b1, W2, b2)


# speedup vs baseline: 1.1562x; 1.1562x over previous
import functools

import jax
import jax.numpy as jnp
from jax import lax
from jax.experimental import pallas as pl
from jax.experimental.pallas import tpu as pltpu
from jax.experimental.pallas import tpu_sc as plsc

B = 4096
S = 200
D = 64
L = 16
CHUNK = 40
CPR = S // CHUNK
NW = 32
BPW = B // NW
TASKS = BPW * CPR
DV = D // L
NBUF = 4
RB = 4000


def _widen_body(x_ref, o_ref):
    o_ref[:, : D] = x_ref[...]


def _widen(emb):
    v = emb.shape[0]
    return pl.pallas_call(
        _widen_body,
        grid=(v // RB,),
        in_specs=[pl.BlockSpec((RB, D), lambda i: (i, 0))],
        out_specs=pl.BlockSpec((RB, 2 * D), lambda i: (i, 0)),
        out_shape=jax.ShapeDtypeStruct((v, 2 * D), jnp.float32),
    )(emb)


def _make_pool(num_tables):
    owidth = num_tables * D

    def body(*refs):
        toks = refs[:num_tables]
        embs = refs[num_tables:2 * num_tables]
        out = refs[2 * num_tables]
        idx_v = refs[2 * num_tables + 1]
        rbufs = refs[2 * num_tables + 2:2 * num_tables + 2 + NBUF]
        stage = refs[2 * num_tables + 2 + NBUF]
        sem = refs[2 * num_tables + 3 + NBUF]

        cid = lax.axis_index("c")
        sid = lax.axis_index("s")
        wid = sid * 2 + cid

        def zbody(i, carry):
            z = jnp.zeros((L,), jnp.float32)
            for j in range(owidth // L):
                stage[i, pl.ds(L * j, L)] = z
            return carry

        lax.fori_loop(0, BPW, zbody, 0)

        for t in range(num_tables):
            tok = toks[t]
            emb = embs[t]
            pltpu.sync_copy(tok.at[pl.ds(wid * TASKS, TASKS)], idx_v)

            def fire(k, rbuf, emb=emb):
                pltpu.make_async_copy(emb.at[idx_v.at[k]], rbuf, sem).start()

            def drain(k, rbuf, emb=emb):
                pltpu.make_async_copy(emb.at[idx_v.at[k]], rbuf, sem).wait()

            def accum(k, rbuf, t=t):
                acc = [jnp.zeros((L,), jnp.float32) for _ in range(2 * DV)]
                for s in range(CHUNK):
                    bank = (s % 2) * DV
                    for j in range(DV):
                        acc[bank + j] = acc[bank + j] + rbuf[s, pl.ds(L * j, L)]
                b_loc = k // CPR
                for j in range(DV):
                    plsc.addupdate(
                        stage.at[b_loc, pl.ds(t * D + L * j, L)],
                        acc[j] + acc[DV + j],
                    )

            for p in range(NBUF - 1):
                fire(p, rbufs[p])

            def lbody(kk, carry):
                for p in range(NBUF):
                    k = NBUF * kk + p

                    drain(k, rbufs[p])

                    @pl.when(k + NBUF - 1 < TASKS)
                    def _(k=k, p=p):
                        fire(k + NBUF - 1, rbufs[(p + NBUF - 1) % NBUF])

                    accum(k, rbufs[p])
                return carry

            lax.fori_loop(0, TASKS // NBUF, lbody, 0)

        pltpu.sync_copy(stage, out.at[pl.ds(wid * BPW, BPW)])

    return functools.partial(
        pl.kernel,
        out_type=jax.ShapeDtypeStruct((B, owidth), jnp.float32),
        mesh=plsc.VectorSubcoreMesh(core_axis_name="c", subcore_axis_name="s"),
        scratch_types=(
            [pltpu.VMEM((TASKS, CHUNK), jnp.int32)]
            + [pltpu.VMEM((CHUNK, 2 * D), jnp.float32) for _ in range(NBUF)]
            + [pltpu.VMEM((BPW, owidth), jnp.float32), pltpu.SemaphoreType.DMA]
        ),
    )(body)


_pool1 = _make_pool(1)
_pool2 = _make_pool(2)


def _mlp_body(x1_ref, x23_ref, w1_ref, b1_ref, w2_ref, b2_ref, o_ref):
    h = lax.dot_general(
        x1_ref[...], w1_ref[pl.ds(0, D), :], (((1,), (0,)), ((), ())),
        preferred_element_type=jnp.float32, precision=lax.Precision.HIGHEST,
    )
    h = h + lax.dot_general(
        x23_ref[...], w1_ref[pl.ds(D, 2 * D), :], (((1,), (0,)), ((), ())),
        preferred_element_type=jnp.float32, precision=lax.Precision.HIGHEST,
    )
    h = jnp.maximum(h * (1.0 / S) + b1_ref[...], 0.0)
    o = lax.dot_general(
        h, w2_ref[...], (((1,), (0,)), ((), ())),
        preferred_element_type=jnp.float32, precision=lax.Precision.HIGHEST,
    )
    o_ref[...] = o + b2_ref[...]


def _mlp(x1, x23, W1, b1, W2, b2):
    return pl.pallas_call(
        _mlp_body,
        out_shape=jax.ShapeDtypeStruct((B, W2.shape[1]), jnp.float32),
    )(x1, x23, W1, b1.reshape(1, -1), W2, b2.reshape(1, -1))


def kernel(tokens_1gram, tokens_2gram, tokens_3gram, emb1, emb2, emb3, W1, b1, W2, b2):
    t1 = tokens_1gram.reshape(-1, CHUNK)
    t2 = tokens_2gram.reshape(-1, CHUNK)
    t3 = tokens_3gram.reshape(-1, CHUNK)
    p2 = _widen(emb2)
    p3 = _widen(emb3)
    p1 = _widen(emb1)
    pooled23 = _pool2(t2, t3, p2, p3)
    pooled1 = _pool1(t1, p1)
    return _mlp(pooled1, pooled23, W1, b1, W2, b2)


# speedup vs baseline: 1.3522x; 1.1695x over previous
import functools

import jax
import jax.numpy as jnp
from jax import lax
from jax.experimental import pallas as pl
from jax.experimental.pallas import tpu as pltpu
from jax.experimental.pallas import tpu_sc as plsc

B = 4096
S = 200
D = 64
L = 16
CHUNK = 40
CPR = S // CHUNK
NW = 32
BPW = B // NW
TASKS = BPW * CPR
DV = D // L
NBUF = 4
RB = 4000


def _widen(emb):
    return jnp.pad(emb, ((0, 0), (0, D)))


def _make_pool(num_tables):
    owidth = num_tables * D

    def body(*refs):
        toks = refs[:num_tables]
        embs = refs[num_tables:2 * num_tables]
        out = refs[2 * num_tables]
        idx_v = refs[2 * num_tables + 1]
        rbufs = refs[2 * num_tables + 2:2 * num_tables + 2 + NBUF]
        stage = refs[2 * num_tables + 2 + NBUF]
        sem = refs[2 * num_tables + 3 + NBUF]

        cid = lax.axis_index("c")
        sid = lax.axis_index("s")
        wid = sid * 2 + cid

        def zbody(i, carry):
            z = jnp.zeros((L,), jnp.float32)
            for j in range(owidth // L):
                stage[i, pl.ds(L * j, L)] = z
            return carry

        lax.fori_loop(0, BPW, zbody, 0)

        for t in range(num_tables):
            tok = toks[t]
            emb = embs[t]
            pltpu.sync_copy(tok.at[pl.ds(wid * TASKS, TASKS)], idx_v)

            def fire(k, rbuf, emb=emb):
                pltpu.make_async_copy(emb.at[idx_v.at[k]], rbuf, sem).start()

            def drain(k, rbuf, emb=emb):
                pltpu.make_async_copy(emb.at[idx_v.at[k]], rbuf, sem).wait()

            def accum(k, rbuf, t=t):
                acc = [jnp.zeros((L,), jnp.float32) for _ in range(2 * DV)]
                for s in range(CHUNK):
                    bank = (s % 2) * DV
                    for j in range(DV):
                        acc[bank + j] = acc[bank + j] + rbuf[s, pl.ds(L * j, L)]
                b_loc = k // CPR
                for j in range(DV):
                    plsc.addupdate(
                        stage.at[b_loc, pl.ds(t * D + L * j, L)],
                        acc[j] + acc[DV + j],
                    )

            for p in range(NBUF - 1):
                fire(p, rbufs[p])

            def lbody(kk, carry):
                for p in range(NBUF):
                    k = NBUF * kk + p

                    drain(k, rbufs[p])

                    @pl.when(k + NBUF - 1 < TASKS)
                    def _(k=k, p=p):
                        fire(k + NBUF - 1, rbufs[(p + NBUF - 1) % NBUF])

                    accum(k, rbufs[p])
                return carry

            lax.fori_loop(0, TASKS // NBUF, lbody, 0)

        pltpu.sync_copy(stage, out.at[pl.ds(wid * BPW, BPW)])

    return functools.partial(
        pl.kernel,
        out_type=jax.ShapeDtypeStruct((B, owidth), jnp.float32),
        mesh=plsc.VectorSubcoreMesh(core_axis_name="c", subcore_axis_name="s"),
        scratch_types=(
            [pltpu.VMEM((TASKS, CHUNK), jnp.int32)]
            + [pltpu.VMEM((CHUNK, 2 * D), jnp.float32) for _ in range(NBUF)]
            + [pltpu.VMEM((BPW, owidth), jnp.float32), pltpu.SemaphoreType.DMA]
        ),
    )(body)


_pool1 = _make_pool(1)
_pool2 = _make_pool(2)


def _mlp_body(x1_ref, x23_ref, w1_ref, b1_ref, w2_ref, b2_ref, o_ref):
    h = lax.dot_general(
        x1_ref[...], w1_ref[pl.ds(0, D), :], (((1,), (0,)), ((), ())),
        preferred_element_type=jnp.float32, precision=lax.Precision.HIGHEST,
    )
    h = h + lax.dot_general(
        x23_ref[...], w1_ref[pl.ds(D, 2 * D), :], (((1,), (0,)), ((), ())),
        preferred_element_type=jnp.float32, precision=lax.Precision.HIGHEST,
    )
    h = jnp.maximum(h * (1.0 / S) + b1_ref[...], 0.0)
    o = lax.dot_general(
        h, w2_ref[...], (((1,), (0,)), ((), ())),
        preferred_element_type=jnp.float32, precision=lax.Precision.HIGHEST,
    )
    o_ref[...] = o + b2_ref[...]


def _mlp(x1, x23, W1, b1, W2, b2):
    return pl.pallas_call(
        _mlp_body,
        out_shape=jax.ShapeDtypeStruct((B, W2.shape[1]), jnp.float32),
    )(x1, x23, W1, b1.reshape(1, -1), W2, b2.reshape(1, -1))


def kernel(tokens_1gram, tokens_2gram, tokens_3gram, emb1, emb2, emb3, W1, b1, W2, b2):
    t1 = tokens_1gram.reshape(-1, CHUNK)
    t2 = tokens_2gram.reshape(-1, CHUNK)
    t3 = tokens_3gram.reshape(-1, CHUNK)
    p2 = _widen(emb2)
    p3 = _widen(emb3)
    p1 = _widen(emb1)
    pooled23 = _pool2(t2, t3, p2, p3)
    pooled1 = _pool1(t1, p1)
    return _mlp(pooled1, pooled23, W1, b1, W2, b2)
